# baseline (speedup 1.0000x reference)
import functools

import jax
import jax.numpy as jnp
from jax import lax
from jax.experimental import pallas as pl
from jax.experimental.pallas import tpu as pltpu
from jax.experimental.pallas import tpu_sc as plsc

B, T, M, K = 32, 2048, 80, 8
L = 16
NMID = K - 1
BSC = 16
BTC = B - BSC
MH = M // 2
CHUNK_M = 8
NCHUNKS = MH // CHUNK_M
UNROLL = 8
NBLK = T // (L * UNROLL)


def _discretize_sc(x_hbm, tab_hbm, out_hbm, xbuf0, xbuf1, obuf0, obuf1,
                   tbuf, in_sem, out_sem):
    xbufs = (xbuf0, xbuf1)
    obufs = (obuf0, obuf1)
    nc = 2
    wid = lax.axis_index("s") * nc + lax.axis_index("c")
    b = wid // 2
    mbase = (wid % 2) * MH

    pltpu.sync_copy(tab_hbm, tbuf)

    def in_copy(g, slot):
        return pltpu.make_async_copy(
            x_hbm.at[b, pl.ds(mbase + g * CHUNK_M, CHUNK_M), :], xbufs[slot],
            in_sem,
        )

    def out_copy(g, slot):
        return pltpu.make_async_copy(
            obufs[slot], out_hbm.at[b, pl.ds(mbase + g * CHUNK_M, CHUNK_M), :],
            out_sem,
        )

    def compute(g, xs, os):
        def chan_body(u, _):
            m = mbase + g * CHUNK_M + u
            tv = tbuf[m, :]
            sv = [jnp.broadcast_to(tv[k], (L,)) for k in range(K)]
            mv = [jnp.broadcast_to(tv[K + i], (L,)) for i in range(NMID)]

            def blk_body(blk, _):
                off = blk * (L * UNROLL)
                for w in range(UNROLL):
                    x = xs[u, pl.ds(off + w * L, L)]
                    v = sv[0]
                    for i in range(NMID):
                        v = jnp.where(x > mv[i], sv[i + 1], v)
                    os[u, pl.ds(off + w * L, L)] = v
                return 0

            lax.fori_loop(0, NBLK, blk_body, 0)
            return 0

        lax.fori_loop(0, CHUNK_M, chan_body, 0)

    in_copy(0, 0).start()
    for g in range(NCHUNKS):
        if g + 1 < NCHUNKS:
            in_copy(g + 1, (g + 1) % 2).start()
        in_copy(g, g % 2).wait()
        compute(g, xbufs[g % 2], obufs[g % 2])
        if g >= 2:
            out_copy(g - 2, g % 2).wait()
        out_copy(g, g % 2).start()
    out_copy(NCHUNKS - 2, (NCHUNKS - 2) % 2).wait()
    out_copy(NCHUNKS - 1, (NCHUNKS - 1) % 2).wait()


def _discretize_tc(x_ref, tab_ref, o_ref):
    x = x_ref[0]
    tab = tab_ref[...]
    sv = [tab[:, k:k + 1] for k in range(K)]
    mv = [tab[:, K + i:K + i + 1] for i in range(NMID)]
    v = jnp.broadcast_to(sv[0], x.shape)
    for i in range(NMID):
        v = jnp.where(x > mv[i], jnp.broadcast_to(sv[i + 1], x.shape), v)
    o_ref[0] = v


@jax.jit
def kernel(melspecs, centroids):
    xt = jnp.transpose(melspecs, (0, 2, 1))
    scs = jnp.sort(centroids, axis=1)
    mids = 0.5 * (scs[:, :-1] + scs[:, 1:])
    tab = jnp.concatenate(
        [scs, mids, jnp.zeros((M, 1), jnp.float32)], axis=1
    )

    mesh = plsc.VectorSubcoreMesh(
        core_axis_name="c", subcore_axis_name="s", num_cores=2, num_subcores=16
    )
    out_sc = pl.kernel(
        _discretize_sc,
        out_type=jax.ShapeDtypeStruct((BSC, M, T), jnp.float32),
        mesh=mesh,
        compiler_params=pltpu.CompilerParams(
            needs_layout_passes=False,
            use_tc_tiling_on_sc=True,
            disable_bounds_checks=True,
            disable_semaphore_checks=True,
        ),
        scratch_types=[
            pltpu.VMEM((CHUNK_M, T), jnp.float32),
            pltpu.VMEM((CHUNK_M, T), jnp.float32),
            pltpu.VMEM((CHUNK_M, T), jnp.float32),
            pltpu.VMEM((CHUNK_M, T), jnp.float32),
            pltpu.VMEM((M, 16), jnp.float32),
            pltpu.SemaphoreType.DMA,
            pltpu.SemaphoreType.DMA,
        ],
    )(xt, tab)

    out_tc = pl.pallas_call(
        _discretize_tc,
        out_shape=jax.ShapeDtypeStruct((BTC, M, T), jnp.float32),
        grid=(BTC,),
        in_specs=[
            pl.BlockSpec((1, M, T), lambda i: (i + BSC, 0, 0)),
            pl.BlockSpec((M, 16), lambda i: (0, 0)),
        ],
        out_specs=pl.BlockSpec((1, M, T), lambda i: (i, 0, 0)),
    )(xt, tab)

    out_t = jnp.concatenate([out_sc, out_tc], axis=0)
    return jnp.transpose(out_t, (0, 2, 1))


# speedup vs baseline: 1.5181x; 1.5181x over previous
import functools

import jax
import jax.numpy as jnp
from jax import lax
from jax.experimental import pallas as pl
from jax.experimental.pallas import tpu as pltpu
from jax.experimental.pallas import tpu_sc as plsc

B, T, M, K = 32, 2048, 80, 8
L = 16
NMID = K - 1
NW = 32
CHUNK_M = 8
NCHUNKS = M // CHUNK_M
UNROLL = 8
NBLK = T // (L * UNROLL)


def _discretize(x_hbm, tab_hbm, out_hbm, xbuf0, xbuf1, obuf0, obuf1,
                tbuf, in_sem, out_sem):
    xbufs = (xbuf0, xbuf1)
    obufs = (obuf0, obuf1)
    nc = 2
    wid = lax.axis_index("s") * nc + lax.axis_index("c")

    pltpu.sync_copy(tab_hbm, tbuf)

    def in_copy(g, slot):
        return pltpu.make_async_copy(
            x_hbm.at[wid, pl.ds(g * CHUNK_M, CHUNK_M), :], xbufs[slot],
            in_sem,
        )

    def out_copy(g, slot):
        return pltpu.make_async_copy(
            obufs[slot], out_hbm.at[wid, pl.ds(g * CHUNK_M, CHUNK_M), :],
            out_sem,
        )

    def compute(g, xs, os):
        for u in range(CHUNK_M):
            m = g * CHUNK_M + u
            tv = tbuf[m, :]
            sv = [jnp.broadcast_to(tv[k], (L,)) for k in range(K)]
            mv = [jnp.broadcast_to(tv[K + i], (L,)) for i in range(NMID)]

            def blk_body(blk, _, u=u, sv=sv, mv=mv):
                off = blk * (L * UNROLL)
                for w in range(UNROLL):
                    x = xs[u, pl.ds(off + w * L, L)]
                    v = sv[0]
                    for i in range(NMID):
                        v = jnp.where(x > mv[i], sv[i + 1], v)
                    os[u, pl.ds(off + w * L, L)] = v
                return 0

            lax.fori_loop(0, NBLK, blk_body, 0)

    in_copy(0, 0).start()

    def pair_body(g2, _):
        for par in range(2):
            g = g2 * 2 + par
            if par == 0:
                in_copy(g + 1, 1).start()
            else:
                @pl.when(g2 < NCHUNKS // 2 - 1)
                def _():
                    in_copy(g + 1, 0).start()
            in_copy(g, par).wait()
            compute(g, xbufs[par], obufs[par])

            @pl.when(g2 > 0)
            def _():
                out_copy(g - 2, par).wait()

            out_copy(g, par).start()
        return 0

    lax.fori_loop(0, NCHUNKS // 2, pair_body, 0)
    out_copy(NCHUNKS - 2, 0).wait()
    out_copy(NCHUNKS - 1, 1).wait()


@jax.jit
def kernel(melspecs, centroids):
    xt = jnp.transpose(melspecs, (0, 2, 1))
    c = [centroids[:, k] for k in range(K)]
    net = [(0, 1), (2, 3), (4, 5), (6, 7),
           (0, 2), (1, 3), (4, 6), (5, 7),
           (1, 2), (5, 6),
           (0, 4), (1, 5), (2, 6), (3, 7),
           (2, 4), (3, 5),
           (1, 2), (3, 4), (5, 6)]
    for a, bb in net:
        lo = jnp.minimum(c[a], c[bb])
        hi = jnp.maximum(c[a], c[bb])
        c[a], c[bb] = lo, hi
    mids = [0.5 * (c[i] + c[i + 1]) for i in range(NMID)]
    tab = jnp.stack(
        c + mids + [jnp.zeros((M,), jnp.float32)], axis=1
    )

    mesh = plsc.VectorSubcoreMesh(
        core_axis_name="c", subcore_axis_name="s", num_cores=2, num_subcores=16
    )
    out_t = pl.kernel(
        _discretize,
        out_type=jax.ShapeDtypeStruct((B, M, T), jnp.float32),
        mesh=mesh,
        compiler_params=pltpu.CompilerParams(
            needs_layout_passes=False,
            use_tc_tiling_on_sc=True,
            disable_bounds_checks=True,
            disable_semaphore_checks=True,
        ),
        scratch_types=[
            pltpu.VMEM((CHUNK_M, T), jnp.float32),
            pltpu.VMEM((CHUNK_M, T), jnp.float32),
            pltpu.VMEM((CHUNK_M, T), jnp.float32),
            pltpu.VMEM((CHUNK_M, T), jnp.float32),
            pltpu.VMEM((M, 16), jnp.float32),
            pltpu.SemaphoreType.DMA,
            pltpu.SemaphoreType.DMA,
        ],
    )(xt, tab)
    return jnp.transpose(out_t, (0, 2, 1))
